# baseline (speedup 1.0000x reference)
import jax
import jax.numpy as jnp
from jax.experimental import pallas as pl

_B = 64


def _fill_body(span_ref, out_ref):
    n = span_ref[:, 0]
    r = jax.lax.broadcasted_iota(jnp.int32, (_B, 100, 100), 1)
    out_ref[...] = (r < n[:, None, None]).astype(jnp.float32)


def kernel(tensor_span):
    batch = tensor_span.shape[0]
    grid = batch // _B
    return pl.pallas_call(
        _fill_body,
        grid=(grid,),
        in_specs=[pl.BlockSpec((_B, 2), lambda i: (i, 0))],
        out_specs=pl.BlockSpec((_B, 100, 100), lambda i: (i, 0, 0)),
        out_shape=jax.ShapeDtypeStruct((batch, 100, 100), jnp.float32),
    )(tensor_span)
